# Initial kernel scaffold; baseline (speedup 1.0000x reference)
import jax
import jax.numpy as jnp
from jax.experimental import pallas as pl


def kernel(x, edge_index):
    raise NotImplementedError("write your pallas kernel here")


# speedup vs baseline: 3.5394x; 3.5394x over previous
import functools

import jax
import jax.numpy as jnp
from jax import lax
from jax.experimental import pallas as pl
from jax.experimental.pallas import tpu as pltpu
from jax.experimental.pallas import tpu_sc as plsc

N_NODES = 10000
N_EDGES = 320000
D_FEAT = 128

NC = 2
NS = 16

DHALF = D_FEAT // NC
E_PER_TILE = N_EDGES // NS
BATCH = 80
NBATCH = E_PER_TILE // BATCH
N_PAD = 10240
ROWS_PER_TILE = N_PAD // NS


def _sc_kernel(xl, xr, src, dst, zeros, outl, outr,
               acc, idx_s, idx_d, rows, sem):
  c = lax.axis_index("c")
  s = lax.axis_index("s")
  r0 = s * ROWS_PER_TILE

  pltpu.sync_copy(zeros, acc.at[pl.ds(r0, ROWS_PER_TILE)])
  plsc.subcore_barrier()

  def body(xh):
    def step(b, carry):
      off = s * E_PER_TILE + b * BATCH
      pltpu.sync_copy(src.at[pl.ds(off, BATCH)], idx_s)
      pltpu.sync_copy(dst.at[pl.ds(off, BATCH)], idx_d)
      pltpu.async_copy(xh.at[idx_s], rows, sem).wait()
      pltpu.sync_copy(rows, acc.at[idx_d], add=True)
      return carry
    lax.fori_loop(0, NBATCH, step, 0)

  @pl.when(c == 0)
  def _():
    body(xl)

  @pl.when(c == 1)
  def _():
    body(xr)

  plsc.subcore_barrier()

  @pl.when(c == 0)
  def _():
    pltpu.sync_copy(acc.at[pl.ds(r0, ROWS_PER_TILE)],
                    outl.at[pl.ds(r0, ROWS_PER_TILE)])

  @pl.when(c == 1)
  def _():
    pltpu.sync_copy(acc.at[pl.ds(r0, ROWS_PER_TILE)],
                    outr.at[pl.ds(r0, ROWS_PER_TILE)])


@jax.jit
def _path_add(xl, xr, src, dst, zeros):
  mesh = plsc.VectorSubcoreMesh(core_axis_name="c", subcore_axis_name="s")
  return pl.kernel(
      _sc_kernel,
      out_type=(
          jax.ShapeDtypeStruct((N_PAD, DHALF), jnp.float32),
          jax.ShapeDtypeStruct((N_PAD, DHALF), jnp.float32),
      ),
      mesh=mesh,
      scratch_types=[
          pltpu.VMEM_SHARED((N_PAD, DHALF), jnp.float32),
          pltpu.VMEM((BATCH,), jnp.int32),
          pltpu.VMEM((BATCH,), jnp.int32),
          pltpu.VMEM((BATCH, DHALF), jnp.float32),
          pltpu.SemaphoreType.DMA,
      ],
      compiler_params=pltpu.CompilerParams(use_tc_tiling_on_sc=False),
      name="path_add_sc",
  )(xl, xr, src, dst, zeros)


def kernel(x, edge_index):
  xl = x[:, :DHALF]
  xr = x[:, DHALF:]
  src = edge_index[0]
  dst = edge_index[1]
  zeros = jnp.zeros((ROWS_PER_TILE, DHALF), jnp.float32)
  outl, outr = _path_add(xl, xr, src, dst, zeros)
  return jnp.concatenate([outl[:N_NODES], outr[:N_NODES]], axis=1)


# speedup vs baseline: 7.7516x; 2.1901x over previous
import functools

import jax
import jax.numpy as jnp
from jax import lax
from jax.experimental import pallas as pl
from jax.experimental.pallas import tpu as pltpu
from jax.experimental.pallas import tpu_sc as plsc

N_NODES = 10000
N_EDGES = 320000
D_FEAT = 128

NC = 2
NS = 16

DHALF = D_FEAT // NC
E_PER_TILE = N_EDGES // NS
BATCH = 80
NBATCH = E_PER_TILE // BATCH
N_PAD = 10240
ROWS_PER_TILE = N_PAD // NS


def _sc_kernel(xl, xr, src3, dst3, zeros, outl, outr,
               acc, idx_s, idx_d, rows0, rows1, g0, g1):
  c = lax.axis_index("c")
  s = lax.axis_index("s")
  r0 = s * ROWS_PER_TILE

  pltpu.sync_copy(zeros, acc.at[pl.ds(r0, ROWS_PER_TILE)])
  pltpu.sync_copy(src3.at[s], idx_s)
  pltpu.sync_copy(dst3.at[s], idx_d)
  plsc.subcore_barrier()

  def body(xh):
    def step(i, carry):
      b0 = 2 * i
      b1 = b0 + 1
      d0 = pltpu.async_copy(xh.at[idx_s.at[b0]], rows0, g0)
      d1 = pltpu.async_copy(xh.at[idx_s.at[b1]], rows1, g1)
      d0.wait()
      pltpu.sync_copy(rows0, acc.at[idx_d.at[b0]], add=True)
      d1.wait()
      pltpu.sync_copy(rows1, acc.at[idx_d.at[b1]], add=True)
      return carry
    lax.fori_loop(0, NBATCH // 2, step, 0)

  @pl.when(c == 0)
  def _():
    body(xl)

  @pl.when(c == 1)
  def _():
    body(xr)

  plsc.subcore_barrier()

  @pl.when(c == 0)
  def _():
    pltpu.sync_copy(acc.at[pl.ds(r0, ROWS_PER_TILE)],
                    outl.at[pl.ds(r0, ROWS_PER_TILE)])

  @pl.when(c == 1)
  def _():
    pltpu.sync_copy(acc.at[pl.ds(r0, ROWS_PER_TILE)],
                    outr.at[pl.ds(r0, ROWS_PER_TILE)])


@jax.jit
def _path_add(xl, xr, src, dst, zeros):
  mesh = plsc.VectorSubcoreMesh(core_axis_name="c", subcore_axis_name="s")
  return pl.kernel(
      _sc_kernel,
      out_type=(
          jax.ShapeDtypeStruct((N_PAD, DHALF), jnp.float32),
          jax.ShapeDtypeStruct((N_PAD, DHALF), jnp.float32),
      ),
      mesh=mesh,
      scratch_types=[
          pltpu.VMEM_SHARED((N_PAD, DHALF), jnp.float32),
          pltpu.VMEM((NBATCH, BATCH), jnp.int32),
          pltpu.VMEM((NBATCH, BATCH), jnp.int32),
          pltpu.VMEM((BATCH, DHALF), jnp.float32),
          pltpu.VMEM((BATCH, DHALF), jnp.float32),
          pltpu.SemaphoreType.DMA,
          pltpu.SemaphoreType.DMA,
      ],
      compiler_params=pltpu.CompilerParams(use_tc_tiling_on_sc=False),
      name="path_add_sc",
  )(xl, xr, src, dst, zeros)


def kernel(x, edge_index):
  xl = x[:, :DHALF]
  xr = x[:, DHALF:]
  src3 = edge_index[0].reshape(NS, NBATCH, BATCH)
  dst3 = edge_index[1].reshape(NS, NBATCH, BATCH)
  zeros = jnp.zeros((ROWS_PER_TILE, DHALF), jnp.float32)
  outl, outr = _path_add(xl, xr, src3, dst3, zeros)
  return jnp.concatenate([outl[:N_NODES], outr[:N_NODES]], axis=1)


# speedup vs baseline: 10.3926x; 1.3407x over previous
import functools

import jax
import jax.numpy as jnp
from jax import lax
from jax.experimental import pallas as pl
from jax.experimental.pallas import tpu as pltpu
from jax.experimental.pallas import tpu_sc as plsc

N_NODES = 10000
N_EDGES = 320000
D_FEAT = 128

NC = 2
NS = 16

DHALF = D_FEAT // NC
E_PER_TILE = N_EDGES // NS
BATCH = 80
NBATCH = E_PER_TILE // BATCH
RING = 5
NGROUP = NBATCH // RING
N_PAD = 10240
ROWS_PER_TILE = N_PAD // NS
LAST_ROWS = N_NODES - 15 * ROWS_PER_TILE


def _sc_kernel(xl, xr, src3, dst3, zeros, out,
               acc, idx_s, idx_d, rows, gsem, ssem):
  c = lax.axis_index("c")
  s = lax.axis_index("s")
  r0 = s * ROWS_PER_TILE

  pltpu.sync_copy(zeros, acc.at[pl.ds(r0, ROWS_PER_TILE)])
  pltpu.sync_copy(src3.at[s], idx_s)
  pltpu.sync_copy(dst3.at[s], idx_d)
  plsc.subcore_barrier()

  def body(xh):
    def step(g, carry):
      base = g * RING
      gd = [
          pltpu.async_copy(xh.at[idx_s.at[base + u]], rows[u], gsem[u])
          for u in range(RING)
      ]
      sd = []
      for u in range(RING):
        gd[u].wait()
        sd.append(
            pltpu.async_copy(rows[u], acc.at[idx_d.at[base + u]], ssem[u],
                             add=True))
      for u in range(RING):
        sd[u].wait()
      return carry
    lax.fori_loop(0, NGROUP, step, 0)

  @pl.when(c == 0)
  def _():
    body(xl)

  @pl.when(c == 1)
  def _():
    body(xr)

  plsc.subcore_barrier()

  @pl.when(s < NS - 1)
  def _():
    pltpu.sync_copy(
        acc.at[pl.ds(r0, ROWS_PER_TILE)],
        out.at[pl.ds(r0, ROWS_PER_TILE), pl.ds(c * DHALF, DHALF)],
    )

  @pl.when(s == NS - 1)
  def _():
    pltpu.sync_copy(
        acc.at[pl.ds(r0, LAST_ROWS)],
        out.at[pl.ds(r0, LAST_ROWS), pl.ds(c * DHALF, DHALF)],
    )


@jax.jit
def _path_add(xl, xr, src3, dst3, zeros):
  mesh = plsc.VectorSubcoreMesh(core_axis_name="c", subcore_axis_name="s")
  return pl.kernel(
      _sc_kernel,
      out_type=jax.ShapeDtypeStruct((N_NODES, D_FEAT), jnp.float32),
      mesh=mesh,
      scratch_types=[
          pltpu.VMEM_SHARED((N_PAD, DHALF), jnp.float32),
          pltpu.VMEM((NBATCH, BATCH), jnp.int32),
          pltpu.VMEM((NBATCH, BATCH), jnp.int32),
          [pltpu.VMEM((BATCH, DHALF), jnp.float32)
           for _ in range(RING)],
          [pltpu.SemaphoreType.DMA for _ in range(RING)],
          [pltpu.SemaphoreType.DMA for _ in range(RING)],
      ],
      compiler_params=pltpu.CompilerParams(use_tc_tiling_on_sc=False),
      name="path_add_sc",
  )(xl, xr, src3, dst3, zeros)


def kernel(x, edge_index):
  xl = x[:, :DHALF]
  xr = x[:, DHALF:]
  src3 = edge_index[0].reshape(NS, NBATCH, BATCH)
  dst3 = edge_index[1].reshape(NS, NBATCH, BATCH)
  zeros = jnp.zeros((ROWS_PER_TILE, DHALF), jnp.float32)
  return _path_add(xl, xr, src3, dst3, zeros)


# speedup vs baseline: 10.6331x; 1.0231x over previous
import functools

import jax
import jax.numpy as jnp
from jax import lax
from jax.experimental import pallas as pl
from jax.experimental.pallas import tpu as pltpu
from jax.experimental.pallas import tpu_sc as plsc

N_NODES = 10000
N_EDGES = 320000
D_FEAT = 128

NC = 2
NS = 16

DHALF = D_FEAT // NC
E_PER_TILE = N_EDGES // NS
BATCH = 160
NBATCH = E_PER_TILE // BATCH
RING = 4
NGROUP = NBATCH // RING
NLEFT = NBATCH - NGROUP * RING
N_PAD = 10240
ROWS_PER_TILE = N_PAD // NS
LAST_ROWS = N_NODES - 15 * ROWS_PER_TILE


def _sc_kernel(xl, xr, src3, dst3, zeros, out,
               acc, idx_s, idx_d, rows, gsem, ssem):
  c = lax.axis_index("c")
  s = lax.axis_index("s")
  r0 = s * ROWS_PER_TILE

  z = pltpu.async_copy(zeros, acc.at[pl.ds(r0, ROWS_PER_TILE)], gsem[0])
  a = pltpu.async_copy(src3.at[s], idx_s, gsem[1])
  b = pltpu.async_copy(dst3.at[s], idx_d, gsem[2])
  z.wait()
  a.wait()
  b.wait()
  plsc.subcore_barrier()

  def body(xh):
    def group(base, n):
      gd = [
          pltpu.async_copy(xh.at[idx_s.at[base + u]], rows[u], gsem[u])
          for u in range(n)
      ]
      sd = []
      for u in range(n):
        gd[u].wait()
        sd.append(
            pltpu.async_copy(rows[u], acc.at[idx_d.at[base + u]], ssem[u],
                             add=True))
      for u in range(n):
        sd[u].wait()

    def step(g, carry):
      group(g * RING, RING)
      return carry
    lax.fori_loop(0, NGROUP, step, 0)
    if NLEFT:
      group(NGROUP * RING, NLEFT)

  @pl.when(c == 0)
  def _():
    body(xl)

  @pl.when(c == 1)
  def _():
    body(xr)

  plsc.subcore_barrier()

  @pl.when(s < NS - 1)
  def _():
    pltpu.sync_copy(
        acc.at[pl.ds(r0, ROWS_PER_TILE)],
        out.at[pl.ds(r0, ROWS_PER_TILE), pl.ds(c * DHALF, DHALF)],
    )

  @pl.when(s == NS - 1)
  def _():
    pltpu.sync_copy(
        acc.at[pl.ds(r0, LAST_ROWS)],
        out.at[pl.ds(r0, LAST_ROWS), pl.ds(c * DHALF, DHALF)],
    )


@jax.jit
def _path_add(xl, xr, src3, dst3, zeros):
  mesh = plsc.VectorSubcoreMesh(core_axis_name="c", subcore_axis_name="s")
  return pl.kernel(
      _sc_kernel,
      out_type=jax.ShapeDtypeStruct((N_NODES, D_FEAT), jnp.float32),
      mesh=mesh,
      scratch_types=[
          pltpu.VMEM_SHARED((N_PAD, DHALF), jnp.float32),
          pltpu.VMEM((NBATCH, BATCH), jnp.int32),
          pltpu.VMEM((NBATCH, BATCH), jnp.int32),
          [pltpu.VMEM((BATCH, DHALF), jnp.float32)
           for _ in range(RING)],
          [pltpu.SemaphoreType.DMA for _ in range(RING)],
          [pltpu.SemaphoreType.DMA for _ in range(RING)],
      ],
      compiler_params=pltpu.CompilerParams(use_tc_tiling_on_sc=False),
      name="path_add_sc",
  )(xl, xr, src3, dst3, zeros)


def kernel(x, edge_index):
  xl = x[:, :DHALF]
  xr = x[:, DHALF:]
  src3 = edge_index[0].reshape(NS, NBATCH, BATCH)
  dst3 = edge_index[1].reshape(NS, NBATCH, BATCH)
  zeros = jnp.zeros((ROWS_PER_TILE, DHALF), jnp.float32)
  return _path_add(xl, xr, src3, dst3, zeros)
